# Initial kernel scaffold; baseline (speedup 1.0000x reference)
import jax
import jax.numpy as jnp
from jax.experimental import pallas as pl


def kernel(inputs, pos_encoding):
    raise NotImplementedError("write your pallas kernel here")


# speedup vs baseline: 1.6769x; 1.6769x over previous
import jax
import jax.numpy as jnp
from jax.experimental import pallas as pl


_SBLK = 512


def _body(x_ref, p_ref, o_ref):
    o_ref[0, :, :] = x_ref[0, :, :] + p_ref[...]


def kernel(inputs, pos_encoding):
    B, S, D = inputs.shape
    pos = pos_encoding[:S]
    grid = (S // _SBLK, B)
    return pl.pallas_call(
        _body,
        grid=grid,
        in_specs=[
            pl.BlockSpec((1, _SBLK, D), lambda s, b: (b, s, 0)),
            pl.BlockSpec((_SBLK, D), lambda s, b: (s, 0)),
        ],
        out_specs=pl.BlockSpec((1, _SBLK, D), lambda s, b: (b, s, 0)),
        out_shape=jax.ShapeDtypeStruct((B, S, D), inputs.dtype),
    )(inputs, pos)


# speedup vs baseline: 1.8480x; 1.1021x over previous
import jax
import jax.numpy as jnp
from jax.experimental import pallas as pl


_SBLK = 1024


def _body(x_ref, p_ref, o_ref):
    o_ref[0, :, :] = x_ref[0, :, :] + p_ref[...]


def kernel(inputs, pos_encoding):
    B, S, D = inputs.shape
    pos = pos_encoding[:S]
    grid = (S // _SBLK, B)
    return pl.pallas_call(
        _body,
        grid=grid,
        in_specs=[
            pl.BlockSpec((1, _SBLK, D), lambda s, b: (b, s, 0)),
            pl.BlockSpec((_SBLK, D), lambda s, b: (s, 0)),
        ],
        out_specs=pl.BlockSpec((1, _SBLK, D), lambda s, b: (b, s, 0)),
        out_shape=jax.ShapeDtypeStruct((B, S, D), inputs.dtype),
    )(inputs, pos)


# speedup vs baseline: 1.9706x; 1.0663x over previous
import jax
import jax.numpy as jnp
from jax.experimental import pallas as pl


_SBLK = 2048


def _body(x_ref, p_ref, o_ref):
    o_ref[0, :, :] = x_ref[0, :, :] + p_ref[...]


def kernel(inputs, pos_encoding):
    B, S, D = inputs.shape
    pos = pos_encoding[:S]
    grid = (S // _SBLK, B)
    return pl.pallas_call(
        _body,
        grid=grid,
        in_specs=[
            pl.BlockSpec((1, _SBLK, D), lambda s, b: (b, s, 0)),
            pl.BlockSpec((_SBLK, D), lambda s, b: (s, 0)),
        ],
        out_specs=pl.BlockSpec((1, _SBLK, D), lambda s, b: (b, s, 0)),
        out_shape=jax.ShapeDtypeStruct((B, S, D), inputs.dtype),
    )(inputs, pos)
